# baseline (speedup 1.0000x reference)
import functools

import jax
import jax.numpy as jnp
from jax.experimental import pallas as pl
from jax.experimental.pallas import tpu as pltpu
from jax.experimental.pallas import tpu_sc as plsc

RANKING_K = 5
TOP_RANKED_N = 1024
TOP_N_SIM = 7
TAU = 0.05
NUM_CLASSES = 65

N_SRC = 8192
N_TGT = 2048
FEAT = 1024
ROW_BLK = 1024
COL_BLK = 256
COL_TILE = 256
N_TILES = N_TGT // COL_TILE
C_PAD = 128

NEG = -3.0


_SC_NW = 32
_SC_L = 16
_SC_ROWS = N_SRC // _SC_NW


def _sc_onehot_body(lab_hbm, out_hbm, lab_v, rows_v):
    c = jax.lax.axis_index("c")
    s = jax.lax.axis_index("s")
    wid = c * 16 + s
    base = wid * _SC_ROWS
    pltpu.sync_copy(lab_hbm.at[pl.ds(base, _SC_ROWS)], lab_v)
    lanes = jax.lax.iota(jnp.int32, _SC_L)

    def body(r, carry):
        lab_b = plsc.load_gather(lab_v, [jnp.full((_SC_L,), r, jnp.int32)])
        for k in range(C_PAD // _SC_L):
            oh = (lanes + (k * _SC_L) == lab_b).astype(jnp.float32)
            rows_v[r, pl.ds(k * _SC_L, _SC_L)] = oh
        return carry

    jax.lax.fori_loop(0, _SC_ROWS, body, 0)
    pltpu.sync_copy(rows_v, out_hbm.at[pl.ds(base, _SC_ROWS)])


@functools.cache
def _sc_onehot_kernel():
    return pl.kernel(
        _sc_onehot_body,
        out_type=jax.ShapeDtypeStruct((N_SRC, C_PAD), jnp.float32),
        mesh=plsc.VectorSubcoreMesh(core_axis_name="c", subcore_axis_name="s"),
        scratch_types=[
            pltpu.VMEM((_SC_ROWS,), jnp.int32),
            pltpu.VMEM((_SC_ROWS, C_PAD), jnp.float32),
        ],
        compiler_params=pltpu.CompilerParams(needs_layout_passes=False),
    )


def _sc_onehot(labels):
    return _sc_onehot_kernel()(labels)


def _matmul_body(s_ref, t_ref, o_ref, sn_ref):
    j = pl.program_id(1)

    @pl.when(j == 0)
    def _():
        s = s_ref[...]
        n2 = jnp.sum(s * s, axis=1, keepdims=True)
        sn_ref[...] = s / jnp.maximum(jnp.sqrt(n2), 1e-12)

    t = t_ref[...]
    tn2 = jnp.sum(t * t, axis=1, keepdims=True)
    tn = t / jnp.maximum(jnp.sqrt(tn2), 1e-12)
    o_ref[...] = jax.lax.dot_general(
        sn_ref[...], tn, (((1,), (1,)), ((), ())),
        preferred_element_type=jnp.float32,
        precision=jax.lax.Precision.DEFAULT,
    )


def _reduce_body(sim_ref, oh_ref, rank_ref, con_ref, work_ref, mask_ref):
    sim = sim_ref[...]

    rows = jax.lax.broadcasted_iota(jnp.int32, (N_SRC, COL_TILE), 0)
    work_ref[...] = sim
    mask_ref[...] = jnp.zeros((N_SRC, COL_TILE), jnp.float32)
    top1 = None
    for k in range(TOP_N_SIM):
        w = work_ref[...]
        m = jnp.max(w, axis=0, keepdims=True)
        if k == 0:
            top1 = m
        idx = jnp.min(jnp.where(w == m, rows, N_SRC), axis=0, keepdims=True)
        hit = rows == idx
        work_ref[...] = jnp.where(hit, NEG, w)
        mask_ref[...] = mask_ref[...] + hit.astype(jnp.float32)

    onehot_l = oh_ref[...]
    counts = jax.lax.dot_general(
        mask_ref[...], onehot_l, (((0,), (0,)), ((), ())),
        preferred_element_type=jnp.float32,
    )
    cmax = jnp.max(counts, axis=1, keepdims=True)
    classes_ct = jax.lax.broadcasted_iota(jnp.int32, (COL_TILE, C_PAD), 1)
    assigned = jnp.min(
        jnp.where(counts == cmax, classes_ct, C_PAD), axis=1, keepdims=True
    )
    onehot_a = (assigned == classes_ct).astype(jnp.float32)

    posf = jax.lax.dot_general(
        onehot_l, onehot_a, (((1,), (1,)), ((), ())),
        preferred_element_type=jnp.float32,
    )
    mask_ref[...] = posf

    def top5_sum():
        tot = jnp.zeros((1, COL_TILE), jnp.float32)
        rem = jnp.full((1, COL_TILE), float(RANKING_K), jnp.float32)
        for _ in range(RANKING_K):
            w = work_ref[...]
            m = jnp.max(w, axis=0, keepdims=True)
            eqm = w == m
            cnt = jnp.sum(eqm.astype(jnp.float32), axis=0, keepdims=True)
            take = jnp.clip(rem, 0.0, cnt)
            tot = tot + jnp.where(m > -2.0, take * m, 0.0)
            rem = rem - cnt
            work_ref[...] = jnp.where(eqm, NEG, w)
        return tot

    posm = mask_ref[...] > 0.5
    work_ref[...] = jnp.where(posm, sim, NEG)
    pos_sum = top5_sum()
    work_ref[...] = jnp.where(posm, NEG, sim)
    neg_sum = top5_sum()
    rank_ref[...] = pos_sum / neg_sum

    e = jnp.exp((sim - top1) * (1.0 / TAU))
    total = jnp.sum(e, axis=0, keepdims=True)
    pos_e = jnp.sum(e * mask_ref[...], axis=0, keepdims=True)
    con_ref[...] = pos_e / total


def _loss_body(rank_ref, con_ref, loss_ref):
    r_row = rank_ref[...]
    r_col = r_row.reshape(N_TGT, 1)
    j_row = jax.lax.broadcasted_iota(jnp.int32, (1, N_TGT), 1)
    i_col = jax.lax.broadcasted_iota(jnp.int32, (N_TGT, 1), 0)
    beats = jnp.logical_or(
        r_row > r_col, jnp.logical_and(r_row == r_col, j_row < i_col)
    )
    nbeats = jnp.sum(beats.astype(jnp.float32), axis=1, keepdims=True)
    sel = (nbeats < TOP_RANKED_N).astype(jnp.float32)
    c = con_ref[...].reshape(N_TGT, 1)
    loss = -jnp.sum(sel * jnp.log(c + 1e-6), keepdims=True) / TOP_RANKED_N
    loss_ref[...] = loss.reshape(1, 1)


def kernel(source_features, source_labels, target_features):
    onehot_l = _sc_onehot(source_labels.astype(jnp.int32))

    sim = pl.pallas_call(
        _matmul_body,
        grid=(N_SRC // ROW_BLK, N_TGT // COL_BLK),
        in_specs=[
            pl.BlockSpec((ROW_BLK, FEAT), lambda i, j: (i, 0)),
            pl.BlockSpec((COL_BLK, FEAT), lambda i, j: (j, 0)),
        ],
        out_specs=pl.BlockSpec((ROW_BLK, COL_BLK), lambda i, j: (i, j)),
        out_shape=jax.ShapeDtypeStruct((N_SRC, N_TGT), jnp.float32),
        scratch_shapes=[pltpu.VMEM((ROW_BLK, FEAT), jnp.float32)],
        compiler_params=pltpu.CompilerParams(
            dimension_semantics=("arbitrary", "arbitrary"),
        ),
    )(source_features, target_features)

    ranking, contrast = pl.pallas_call(
        _reduce_body,
        grid=(N_TILES,),
        in_specs=[
            pl.BlockSpec((N_SRC, COL_TILE), lambda i: (0, i)),
            pl.BlockSpec((N_SRC, C_PAD), lambda i: (0, 0)),
        ],
        out_specs=[
            pl.BlockSpec((1, COL_TILE), lambda i: (0, i)),
            pl.BlockSpec((1, COL_TILE), lambda i: (0, i)),
        ],
        out_shape=[
            jax.ShapeDtypeStruct((1, N_TGT), jnp.float32),
            jax.ShapeDtypeStruct((1, N_TGT), jnp.float32),
        ],
        scratch_shapes=[
            pltpu.VMEM((N_SRC, COL_TILE), jnp.float32),
            pltpu.VMEM((N_SRC, COL_TILE), jnp.float32),
        ],
        compiler_params=pltpu.CompilerParams(
            dimension_semantics=("arbitrary",),
        ),
    )(sim, onehot_l)

    loss = pl.pallas_call(
        _loss_body,
        in_specs=[
            pl.BlockSpec((1, N_TGT), lambda: (0, 0)),
            pl.BlockSpec((1, N_TGT), lambda: (0, 0)),
        ],
        out_specs=pl.BlockSpec((1, 1), lambda: (0, 0)),
        out_shape=jax.ShapeDtypeStruct((1, 1), jnp.float32),
    )(ranking, contrast)

    return loss[0, 0]


# speedup vs baseline: 1.0506x; 1.0506x over previous
import functools

import jax
import jax.numpy as jnp
from jax.experimental import pallas as pl
from jax.experimental.pallas import tpu as pltpu
from jax.experimental.pallas import tpu_sc as plsc

RANKING_K = 5
TOP_RANKED_N = 1024
TOP_N_SIM = 7
TAU = 0.05
NUM_CLASSES = 65

N_SRC = 8192
N_TGT = 2048
FEAT = 1024
ROW_BLK = 1024
COL_BLK = 256
COL_TILE = 128
N_TILES = N_TGT // COL_TILE
C_PAD = 128

NEG = -3.0


_SC_NW = 32
_SC_L = 16
_SC_ROWS = N_SRC // _SC_NW


def _sc_onehot_body(lab_hbm, out_hbm, lab_v, rows_v):
    c = jax.lax.axis_index("c")
    s = jax.lax.axis_index("s")
    wid = c * 16 + s
    base = wid * _SC_ROWS
    pltpu.sync_copy(lab_hbm.at[pl.ds(base, _SC_ROWS)], lab_v)
    lanes = jax.lax.iota(jnp.int32, _SC_L)

    def body(r, carry):
        lab_b = plsc.load_gather(lab_v, [jnp.full((_SC_L,), r, jnp.int32)])
        for k in range(C_PAD // _SC_L):
            oh = (lanes + (k * _SC_L) == lab_b).astype(jnp.float32)
            rows_v[r, pl.ds(k * _SC_L, _SC_L)] = oh
        return carry

    jax.lax.fori_loop(0, _SC_ROWS, body, 0)
    pltpu.sync_copy(rows_v, out_hbm.at[pl.ds(base, _SC_ROWS)])


@functools.cache
def _sc_onehot_kernel():
    return pl.kernel(
        _sc_onehot_body,
        out_type=jax.ShapeDtypeStruct((N_SRC, C_PAD), jnp.float32),
        mesh=plsc.VectorSubcoreMesh(core_axis_name="c", subcore_axis_name="s"),
        scratch_types=[
            pltpu.VMEM((_SC_ROWS,), jnp.int32),
            pltpu.VMEM((_SC_ROWS, C_PAD), jnp.float32),
        ],
        compiler_params=pltpu.CompilerParams(needs_layout_passes=False),
    )


def _sc_onehot(labels):
    return _sc_onehot_kernel()(labels)


def _matmul_body(s_ref, t_ref, o_ref, sn_ref):
    j = pl.program_id(1)

    @pl.when(j == 0)
    def _():
        s = s_ref[...]
        n2 = jnp.sum(s * s, axis=1, keepdims=True)
        sn_ref[...] = s / jnp.maximum(jnp.sqrt(n2), 1e-12)

    t = t_ref[...]
    tn2 = jnp.sum(t * t, axis=1, keepdims=True)
    tn = t / jnp.maximum(jnp.sqrt(tn2), 1e-12)
    o_ref[...] = jax.lax.dot_general(
        sn_ref[...], tn, (((1,), (1,)), ((), ())),
        preferred_element_type=jnp.float32,
        precision=jax.lax.Precision.DEFAULT,
    )


def _reduce_body(sim_ref, oh_ref, rank_ref, con_ref, work_ref, mask_ref):
    sim = sim_ref[...]

    rows = jax.lax.broadcasted_iota(jnp.int32, (N_SRC, COL_TILE), 0)
    onehot_l = oh_ref[...]
    work_ref[...] = sim
    counts = jnp.zeros((COL_TILE, C_PAD), jnp.float32)
    top1 = None
    for k in range(TOP_N_SIM):
        w = work_ref[...]
        m = jnp.max(w, axis=0, keepdims=True)
        if k == 0:
            top1 = m
        idx = jnp.min(jnp.where(w == m, rows, N_SRC), axis=0, keepdims=True)
        hit = rows == idx
        hitf = hit.astype(jnp.float32)
        work_ref[...] = jnp.where(hit, NEG, w)
        counts = counts + jax.lax.dot_general(
            hitf, onehot_l, (((0,), (0,)), ((), ())),
            preferred_element_type=jnp.float32,
        )
    cmax = jnp.max(counts, axis=1, keepdims=True)
    classes_ct = jax.lax.broadcasted_iota(jnp.int32, (COL_TILE, C_PAD), 1)
    assigned = jnp.min(
        jnp.where(counts == cmax, classes_ct, C_PAD), axis=1, keepdims=True
    )
    onehot_a = (assigned == classes_ct).astype(jnp.float32)

    posf = jax.lax.dot_general(
        onehot_l, onehot_a, (((1,), (1,)), ((), ())),
        preferred_element_type=jnp.float32,
    )
    mask_ref[...] = posf

    def top5_sum():
        tot = jnp.zeros((1, COL_TILE), jnp.float32)
        rem = jnp.full((1, COL_TILE), float(RANKING_K), jnp.float32)
        for _ in range(RANKING_K):
            w = work_ref[...]
            m = jnp.max(w, axis=0, keepdims=True)
            eqm = w == m
            cnt = jnp.sum(eqm.astype(jnp.float32), axis=0, keepdims=True)
            take = jnp.clip(rem, 0.0, cnt)
            tot = tot + jnp.where(m > -2.0, take * m, 0.0)
            rem = rem - cnt
            work_ref[...] = jnp.where(eqm, NEG, w)
        return tot

    posm = mask_ref[...] > 0.5
    work_ref[...] = jnp.where(posm, sim, NEG)
    pos_sum = top5_sum()
    work_ref[...] = jnp.where(posm, NEG, sim)
    neg_sum = top5_sum()
    rank_ref[...] = pos_sum / neg_sum

    e = jnp.exp((sim - top1) * (1.0 / TAU))
    total = jnp.sum(e, axis=0, keepdims=True)
    pos_e = jnp.sum(e * mask_ref[...], axis=0, keepdims=True)
    con_ref[...] = pos_e / total


def _loss_body(rank_ref, con_ref, loss_ref):
    r_row = rank_ref[...]
    r_col = r_row.reshape(N_TGT, 1)
    j_row = jax.lax.broadcasted_iota(jnp.int32, (1, N_TGT), 1)
    i_col = jax.lax.broadcasted_iota(jnp.int32, (N_TGT, 1), 0)
    beats = jnp.logical_or(
        r_row > r_col, jnp.logical_and(r_row == r_col, j_row < i_col)
    )
    nbeats = jnp.sum(beats.astype(jnp.float32), axis=1, keepdims=True)
    sel = (nbeats < TOP_RANKED_N).astype(jnp.float32)
    c = con_ref[...].reshape(N_TGT, 1)
    loss = -jnp.sum(sel * jnp.log(c + 1e-6), keepdims=True) / TOP_RANKED_N
    loss_ref[...] = loss.reshape(1, 1)


def kernel(source_features, source_labels, target_features):
    onehot_l = _sc_onehot(source_labels.astype(jnp.int32))

    sim = pl.pallas_call(
        _matmul_body,
        grid=(N_SRC // ROW_BLK, N_TGT // COL_BLK),
        in_specs=[
            pl.BlockSpec((ROW_BLK, FEAT), lambda i, j: (i, 0)),
            pl.BlockSpec((COL_BLK, FEAT), lambda i, j: (j, 0)),
        ],
        out_specs=pl.BlockSpec((ROW_BLK, COL_BLK), lambda i, j: (i, j)),
        out_shape=jax.ShapeDtypeStruct((N_SRC, N_TGT), jnp.float32),
        scratch_shapes=[pltpu.VMEM((ROW_BLK, FEAT), jnp.float32)],
        compiler_params=pltpu.CompilerParams(
            dimension_semantics=("arbitrary", "arbitrary"),
        ),
    )(source_features, target_features)

    ranking, contrast = pl.pallas_call(
        _reduce_body,
        grid=(N_TILES,),
        in_specs=[
            pl.BlockSpec((N_SRC, COL_TILE), lambda i: (0, i)),
            pl.BlockSpec((N_SRC, C_PAD), lambda i: (0, 0)),
        ],
        out_specs=[
            pl.BlockSpec((1, COL_TILE), lambda i: (0, i)),
            pl.BlockSpec((1, COL_TILE), lambda i: (0, i)),
        ],
        out_shape=[
            jax.ShapeDtypeStruct((1, N_TGT), jnp.float32),
            jax.ShapeDtypeStruct((1, N_TGT), jnp.float32),
        ],
        scratch_shapes=[
            pltpu.VMEM((N_SRC, COL_TILE), jnp.float32),
            pltpu.VMEM((N_SRC, COL_TILE), jnp.float32),
        ],
        compiler_params=pltpu.CompilerParams(
            dimension_semantics=("arbitrary",),
        ),
    )(sim, onehot_l)

    loss = pl.pallas_call(
        _loss_body,
        in_specs=[
            pl.BlockSpec((1, N_TGT), lambda: (0, 0)),
            pl.BlockSpec((1, N_TGT), lambda: (0, 0)),
        ],
        out_specs=pl.BlockSpec((1, 1), lambda: (0, 0)),
        out_shape=jax.ShapeDtypeStruct((1, 1), jnp.float32),
    )(ranking, contrast)

    return loss[0, 0]


# speedup vs baseline: 1.0521x; 1.0014x over previous
import functools

import jax
import jax.numpy as jnp
from jax.experimental import pallas as pl
from jax.experimental.pallas import tpu as pltpu
from jax.experimental.pallas import tpu_sc as plsc

RANKING_K = 5
TOP_RANKED_N = 1024
TOP_N_SIM = 7
TAU = 0.05
NUM_CLASSES = 65

N_SRC = 8192
N_TGT = 2048
FEAT = 1024
ROW_BLK = 1024
COL_BLK = 256
COL_TILE = 128
N_TILES = N_TGT // COL_TILE
C_PAD = 128

NEG = -3.0


_SC_NW = 32
_SC_L = 16
_SC_ROWS = N_SRC // _SC_NW


def _sc_onehot_body(lab_hbm, out_hbm, lab_v, rows_v):
    c = jax.lax.axis_index("c")
    s = jax.lax.axis_index("s")
    wid = c * 16 + s
    base = wid * _SC_ROWS
    pltpu.sync_copy(lab_hbm.at[pl.ds(base, _SC_ROWS)], lab_v)
    lanes = jax.lax.iota(jnp.int32, _SC_L)

    def body(r, carry):
        lab_b = plsc.load_gather(lab_v, [jnp.full((_SC_L,), r, jnp.int32)])
        for k in range(C_PAD // _SC_L):
            oh = (lanes + (k * _SC_L) == lab_b).astype(jnp.float32)
            rows_v[r, pl.ds(k * _SC_L, _SC_L)] = oh
        return carry

    jax.lax.fori_loop(0, _SC_ROWS, body, 0)
    pltpu.sync_copy(rows_v, out_hbm.at[pl.ds(base, _SC_ROWS)])


@functools.cache
def _sc_onehot_kernel():
    return pl.kernel(
        _sc_onehot_body,
        out_type=jax.ShapeDtypeStruct((N_SRC, C_PAD), jnp.float32),
        mesh=plsc.VectorSubcoreMesh(core_axis_name="c", subcore_axis_name="s"),
        scratch_types=[
            pltpu.VMEM((_SC_ROWS,), jnp.int32),
            pltpu.VMEM((_SC_ROWS, C_PAD), jnp.float32),
        ],
        compiler_params=pltpu.CompilerParams(needs_layout_passes=False),
    )


def _sc_onehot(labels):
    return _sc_onehot_kernel()(labels)


def _matmul_body(s_ref, t_ref, o_ref, sn_ref):
    j = pl.program_id(1)

    @pl.when(j == 0)
    def _():
        s = s_ref[...]
        n2 = jnp.sum(s * s, axis=1, keepdims=True)
        sn_ref[...] = s / jnp.maximum(jnp.sqrt(n2), 1e-12)

    t = t_ref[...]
    tn2 = jnp.sum(t * t, axis=1, keepdims=True)
    tn = t / jnp.maximum(jnp.sqrt(tn2), 1e-12)
    o_ref[...] = jax.lax.dot_general(
        sn_ref[...], tn, (((1,), (1,)), ((), ())),
        preferred_element_type=jnp.float32,
        precision=jax.lax.Precision.DEFAULT,
    )


def _reduce_body(sim_ref, oh_ref, loss_ref, work_ref, mask_ref, rank_scr, lg_scr):
    i = pl.program_id(0)
    sim = sim_ref[...]

    rows = jax.lax.broadcasted_iota(jnp.int32, (N_SRC, COL_TILE), 0)
    onehot_l = oh_ref[...]
    work_ref[...] = sim
    counts = jnp.zeros((COL_TILE, C_PAD), jnp.float32)
    top1 = None
    for k in range(TOP_N_SIM):
        w = work_ref[...]
        m = jnp.max(w, axis=0, keepdims=True)
        if k == 0:
            top1 = m
        idx = jnp.min(jnp.where(w == m, rows, N_SRC), axis=0, keepdims=True)
        hit = rows == idx
        hitf = hit.astype(jnp.float32)
        work_ref[...] = jnp.where(hit, NEG, w)
        counts = counts + jax.lax.dot_general(
            hitf, onehot_l, (((0,), (0,)), ((), ())),
            preferred_element_type=jnp.float32,
        )
    cmax = jnp.max(counts, axis=1, keepdims=True)
    classes_ct = jax.lax.broadcasted_iota(jnp.int32, (COL_TILE, C_PAD), 1)
    assigned = jnp.min(
        jnp.where(counts == cmax, classes_ct, C_PAD), axis=1, keepdims=True
    )
    onehot_a = (assigned == classes_ct).astype(jnp.float32)

    posf = jax.lax.dot_general(
        onehot_l, onehot_a, (((1,), (1,)), ((), ())),
        preferred_element_type=jnp.float32,
    )
    mask_ref[...] = posf

    def top5_sum():
        tot = jnp.zeros((1, COL_TILE), jnp.float32)
        rem = jnp.full((1, COL_TILE), float(RANKING_K), jnp.float32)
        for _ in range(RANKING_K):
            w = work_ref[...]
            m = jnp.max(w, axis=0, keepdims=True)
            eqm = w == m
            cnt = jnp.sum(eqm.astype(jnp.float32), axis=0, keepdims=True)
            take = jnp.clip(rem, 0.0, cnt)
            tot = tot + jnp.where(m > -2.0, take * m, 0.0)
            rem = rem - cnt
            work_ref[...] = jnp.where(eqm, NEG, w)
        return tot

    posm = mask_ref[...] > 0.5
    work_ref[...] = jnp.where(posm, sim, NEG)
    pos_sum = top5_sum()
    work_ref[...] = jnp.where(posm, NEG, sim)
    neg_sum = top5_sum()
    rank_scr[0, pl.ds(i * COL_TILE, COL_TILE)] = (pos_sum / neg_sum)[0, :]

    e = jnp.exp((sim - top1) * (1.0 / TAU))
    total = jnp.sum(e, axis=0, keepdims=True)
    pos_e = jnp.sum(e * mask_ref[...], axis=0, keepdims=True)
    lg_scr[0, pl.ds(i * COL_TILE, COL_TILE)] = jnp.log(pos_e / total + 1e-6)[0, :]

    @pl.when(i == N_TILES - 1)
    def _loss():
        r_row = rank_scr[...]
        r_col = r_row.reshape(N_TGT, 1)
        j_row = jax.lax.broadcasted_iota(jnp.int32, (1, N_TGT), 1)
        i_col = jax.lax.broadcasted_iota(jnp.int32, (N_TGT, 1), 0)
        beats = jnp.logical_or(
            r_row > r_col, jnp.logical_and(r_row == r_col, j_row < i_col)
        )
        nbeats = jnp.sum(beats.astype(jnp.float32), axis=1, keepdims=True)
        sel = (nbeats < TOP_RANKED_N).astype(jnp.float32)
        lg = lg_scr[...].reshape(N_TGT, 1)
        loss = -jnp.sum(sel * lg, keepdims=True) / TOP_RANKED_N
        loss_ref[...] = loss.reshape(1, 1)


def kernel(source_features, source_labels, target_features):
    onehot_l = _sc_onehot(source_labels.astype(jnp.int32))

    sim = pl.pallas_call(
        _matmul_body,
        grid=(N_SRC // ROW_BLK, N_TGT // COL_BLK),
        in_specs=[
            pl.BlockSpec((ROW_BLK, FEAT), lambda i, j: (i, 0)),
            pl.BlockSpec((COL_BLK, FEAT), lambda i, j: (j, 0)),
        ],
        out_specs=pl.BlockSpec((ROW_BLK, COL_BLK), lambda i, j: (i, j)),
        out_shape=jax.ShapeDtypeStruct((N_SRC, N_TGT), jnp.float32),
        scratch_shapes=[pltpu.VMEM((ROW_BLK, FEAT), jnp.float32)],
        compiler_params=pltpu.CompilerParams(
            dimension_semantics=("arbitrary", "arbitrary"),
        ),
    )(source_features, target_features)

    loss = pl.pallas_call(
        _reduce_body,
        grid=(N_TILES,),
        in_specs=[
            pl.BlockSpec((N_SRC, COL_TILE), lambda i: (0, i)),
            pl.BlockSpec((N_SRC, C_PAD), lambda i: (0, 0)),
        ],
        out_specs=pl.BlockSpec((1, 1), lambda i: (0, 0)),
        out_shape=jax.ShapeDtypeStruct((1, 1), jnp.float32),
        scratch_shapes=[
            pltpu.VMEM((N_SRC, COL_TILE), jnp.float32),
            pltpu.VMEM((N_SRC, COL_TILE), jnp.float32),
            pltpu.VMEM((1, N_TGT), jnp.float32),
            pltpu.VMEM((1, N_TGT), jnp.float32),
        ],
        compiler_params=pltpu.CompilerParams(
            dimension_semantics=("arbitrary",),
        ),
    )(sim, onehot_l)

    return loss[0, 0]


# speedup vs baseline: 1.1337x; 1.0776x over previous
import functools

import jax
import jax.numpy as jnp
from jax.experimental import pallas as pl
from jax.experimental.pallas import tpu as pltpu
from jax.experimental.pallas import tpu_sc as plsc

RANKING_K = 5
TOP_RANKED_N = 1024
TOP_N_SIM = 7
TAU = 0.05
NUM_CLASSES = 65

N_SRC = 8192
N_TGT = 2048
FEAT = 1024
ROW_BLK = 2048
COL_BLK = 1024
COL_TILE = 128
N_TILES = N_TGT // COL_TILE
C_PAD = 128

NEG = -3.0


_SC_NW = 32
_SC_L = 16
_SC_ROWS = N_SRC // _SC_NW


def _sc_onehot_body(lab_hbm, out_hbm, lab_v, rows_v):
    c = jax.lax.axis_index("c")
    s = jax.lax.axis_index("s")
    wid = c * 16 + s
    base = wid * _SC_ROWS
    pltpu.sync_copy(lab_hbm.at[pl.ds(base, _SC_ROWS)], lab_v)
    lanes = jax.lax.iota(jnp.int32, _SC_L)

    def body(r, carry):
        lab_b = plsc.load_gather(lab_v, [jnp.full((_SC_L,), r, jnp.int32)])
        for k in range(C_PAD // _SC_L):
            oh = (lanes + (k * _SC_L) == lab_b).astype(jnp.float32)
            rows_v[r, pl.ds(k * _SC_L, _SC_L)] = oh
        return carry

    jax.lax.fori_loop(0, _SC_ROWS, body, 0)
    pltpu.sync_copy(rows_v, out_hbm.at[pl.ds(base, _SC_ROWS)])


@functools.cache
def _sc_onehot_kernel():
    return pl.kernel(
        _sc_onehot_body,
        out_type=jax.ShapeDtypeStruct((N_SRC, C_PAD), jnp.float32),
        mesh=plsc.VectorSubcoreMesh(core_axis_name="c", subcore_axis_name="s"),
        scratch_types=[
            pltpu.VMEM((_SC_ROWS,), jnp.int32),
            pltpu.VMEM((_SC_ROWS, C_PAD), jnp.float32),
        ],
        compiler_params=pltpu.CompilerParams(needs_layout_passes=False),
    )


def _sc_onehot(labels):
    return _sc_onehot_kernel()(labels)


def _matmul_body(s_ref, t_ref, o_ref, sn_ref):
    j = pl.program_id(1)

    @pl.when(j == 0)
    def _():
        s = s_ref[...]
        n2 = jnp.sum(s * s, axis=1, keepdims=True)
        sn_ref[...] = s / jnp.maximum(jnp.sqrt(n2), 1e-12)

    t = t_ref[...]
    tn2 = jnp.sum(t * t, axis=1, keepdims=True)
    tn = t / jnp.maximum(jnp.sqrt(tn2), 1e-12)
    o_ref[...] = jax.lax.dot_general(
        sn_ref[...], tn, (((1,), (1,)), ((), ())),
        preferred_element_type=jnp.float32,
        precision=jax.lax.Precision.DEFAULT,
    )


def _reduce_body(sim_ref, oh_ref, loss_ref, rank_scr, lg_scr):
    i = pl.program_id(0)
    sim = sim_ref[...]

    rows = jax.lax.broadcasted_iota(jnp.int32, (N_SRC, COL_TILE), 0)
    onehot_l = oh_ref[...]
    counts = jnp.zeros((COL_TILE, C_PAD), jnp.float32)
    top1 = None
    w = sim
    for k in range(TOP_N_SIM):
        m = jnp.max(w, axis=0, keepdims=True)
        if k == 0:
            top1 = m
        idx = jnp.min(jnp.where(w == m, rows, N_SRC), axis=0, keepdims=True)
        hit = rows == idx
        hitf = hit.astype(jnp.float32)
        w = jnp.where(hit, NEG, w)
        counts = counts + jax.lax.dot_general(
            hitf, onehot_l, (((0,), (0,)), ((), ())),
            preferred_element_type=jnp.float32,
        )
    cmax = jnp.max(counts, axis=1, keepdims=True)
    classes_ct = jax.lax.broadcasted_iota(jnp.int32, (COL_TILE, C_PAD), 1)
    assigned = jnp.min(
        jnp.where(counts == cmax, classes_ct, C_PAD), axis=1, keepdims=True
    )
    onehot_a = (assigned == classes_ct).astype(jnp.float32)

    posf = jax.lax.dot_general(
        onehot_l, onehot_a, (((1,), (1,)), ((), ())),
        preferred_element_type=jnp.float32,
    )

    def top5_sum(w):
        tot = jnp.zeros((1, COL_TILE), jnp.float32)
        rem = jnp.full((1, COL_TILE), float(RANKING_K), jnp.float32)
        for _ in range(RANKING_K):
            m = jnp.max(w, axis=0, keepdims=True)
            eqm = w == m
            cnt = jnp.sum(eqm.astype(jnp.float32), axis=0, keepdims=True)
            take = jnp.clip(rem, 0.0, cnt)
            tot = tot + jnp.where(m > -2.0, take * m, 0.0)
            rem = rem - cnt
            w = jnp.where(eqm, NEG, w)
        return tot

    posm = posf > 0.5
    pos_sum = top5_sum(jnp.where(posm, sim, NEG))
    neg_sum = top5_sum(jnp.where(posm, NEG, sim))
    rank_scr[0, pl.ds(i * COL_TILE, COL_TILE)] = (pos_sum / neg_sum)[0, :]

    e = jnp.exp((sim - top1) * (1.0 / TAU))
    total = jnp.sum(e, axis=0, keepdims=True)
    pos_e = jnp.sum(e * posf, axis=0, keepdims=True)
    lg_scr[0, pl.ds(i * COL_TILE, COL_TILE)] = jnp.log(pos_e / total + 1e-6)[0, :]

    @pl.when(i == N_TILES - 1)
    def _loss():
        r_row = rank_scr[...]
        r_col = r_row.reshape(N_TGT, 1)
        j_row = jax.lax.broadcasted_iota(jnp.int32, (1, N_TGT), 1)
        i_col = jax.lax.broadcasted_iota(jnp.int32, (N_TGT, 1), 0)
        beats = jnp.logical_or(
            r_row > r_col, jnp.logical_and(r_row == r_col, j_row < i_col)
        )
        nbeats = jnp.sum(beats.astype(jnp.float32), axis=1, keepdims=True)
        sel = (nbeats < TOP_RANKED_N).astype(jnp.float32)
        lg = lg_scr[...].reshape(N_TGT, 1)
        loss = -jnp.sum(sel * lg, keepdims=True) / TOP_RANKED_N
        loss_ref[...] = loss.reshape(1, 1)


def kernel(source_features, source_labels, target_features):
    onehot_l = _sc_onehot(source_labels.astype(jnp.int32))

    sim = pl.pallas_call(
        _matmul_body,
        grid=(N_SRC // ROW_BLK, N_TGT // COL_BLK),
        in_specs=[
            pl.BlockSpec((ROW_BLK, FEAT), lambda i, j: (i, 0)),
            pl.BlockSpec((COL_BLK, FEAT), lambda i, j: (j, 0)),
        ],
        out_specs=pl.BlockSpec((ROW_BLK, COL_BLK), lambda i, j: (i, j)),
        out_shape=jax.ShapeDtypeStruct((N_SRC, N_TGT), jnp.float32),
        scratch_shapes=[pltpu.VMEM((ROW_BLK, FEAT), jnp.float32)],
        compiler_params=pltpu.CompilerParams(
            dimension_semantics=("arbitrary", "arbitrary"),
        ),
    )(source_features, target_features)

    loss = pl.pallas_call(
        _reduce_body,
        grid=(N_TILES,),
        in_specs=[
            pl.BlockSpec((N_SRC, COL_TILE), lambda i: (0, i)),
            pl.BlockSpec((N_SRC, C_PAD), lambda i: (0, 0)),
        ],
        out_specs=pl.BlockSpec((1, 1), lambda i: (0, 0)),
        out_shape=jax.ShapeDtypeStruct((1, 1), jnp.float32),
        scratch_shapes=[
            pltpu.VMEM((1, N_TGT), jnp.float32),
            pltpu.VMEM((1, N_TGT), jnp.float32),
        ],
        compiler_params=pltpu.CompilerParams(
            dimension_semantics=("arbitrary",),
        ),
    )(sim, onehot_l)

    return loss[0, 0]
